# Initial kernel scaffold; baseline (speedup 1.0000x reference)
import jax
import jax.numpy as jnp
from jax.experimental import pallas as pl


def kernel(value, chromosome, hg38_start, hg38_end, W, b, chrom_table):
    raise NotImplementedError("write your pallas kernel here")


# speedup vs baseline: 1.3856x; 1.3856x over previous
import jax
import jax.numpy as jnp
from jax.experimental import pallas as pl

_B, _L, _D, _V = 1024, 200, 128, 25
_TOK_BLK = 2048


def _embed_block(value_ref, chrom_ref, start_ref, end_ref, w_ref, b_ref,
                 denom_ref, table_ref, out_ref):
    v = value_ref[...]
    w = w_ref[...]
    bias = b_ref[...]
    denom = denom_ref[...]
    t = v.shape[0]

    val_emb = v * w + bias

    idx = chrom_ref[...]
    lane = jax.lax.broadcasted_iota(jnp.int32, (t, _V), 1)
    onehot = (idx == lane).astype(jnp.float32)
    chrom_emb = jax.lax.dot_general(
        onehot, table_ref[...], (((1,), (0,)), ((), ())),
        preferred_element_type=jnp.float32)

    even = jax.lax.broadcasted_iota(jnp.int32, (t, _D), 1) % 2 == 0
    a1 = start_ref[...].astype(jnp.float32) / denom
    pe1 = jnp.where(even, jnp.sin(a1), jnp.cos(a1))
    a2 = end_ref[...].astype(jnp.float32) / denom
    pe2 = jnp.where(even, jnp.sin(a2), jnp.cos(a2))

    out_ref[...] = val_emb + chrom_emb + pe1 + pe2


def kernel(value, chromosome, hg38_start, hg38_end, W, b, chrom_table):
    n = _B * _L
    v2 = value.reshape(n, 1)
    c2 = chromosome.reshape(n, 1).astype(jnp.int32)
    s2 = hg38_start.reshape(n, 1).astype(jnp.int32)
    e2 = hg38_end.reshape(n, 1).astype(jnp.int32)
    wrow = W.reshape(1, _D)
    brow = b.reshape(1, _D)
    _2i = jnp.arange(0, _D, 2, dtype=jnp.float32)
    denom = 10000.0 ** (_2i / _D)
    denom_full = jnp.repeat(denom, 2).reshape(1, _D)

    g = n // _TOK_BLK
    tok = pl.BlockSpec((_TOK_BLK, 1), lambda i: (i, 0))
    row = pl.BlockSpec((1, _D), lambda i: (0, 0))
    out = pl.pallas_call(
        _embed_block,
        grid=(g,),
        in_specs=[tok, tok, tok, tok, row, row, row,
                  pl.BlockSpec((_V, _D), lambda i: (0, 0))],
        out_specs=pl.BlockSpec((_TOK_BLK, _D), lambda i: (i, 0)),
        out_shape=jax.ShapeDtypeStruct((n, _D), jnp.float32),
    )(v2, c2, s2, e2, wrow, brow, denom_full, chrom_table)
    return out.reshape(_B, _L, _D)


# speedup vs baseline: 2.5830x; 1.8642x over previous
import jax
import jax.numpy as jnp
from jax.experimental import pallas as pl

_B, _L, _D, _V = 1024, 200, 128, 25
_TOK_BLK = 2048


_TWO_OVER_PI = 0.6366197723675814
_C1H256 = 402.0
_C1H = 1.5703125
_C1L = 4.8387050628662109375e-4
_C2 = -4.371139000186241e-8
_S1, _S2, _S3 = -1.6666654611e-1, 8.3321608736e-3, -1.9515295891e-4
_K1, _K2, _K3 = 4.166664568298827e-2, -1.388731625493765e-3, 2.443315711809948e-5


def _pe_interleaved(pos_f, denom, parity):
    x = pos_f / denom
    kf = jnp.round(x * _TWO_OVER_PI)
    khf = jnp.floor(kf * (1.0 / 256.0))
    klf = kf - khf * 256.0
    d1 = x - khf * _C1H256
    d2 = d1 - klf * _C1H
    d3 = d2 - kf * _C1L
    y = d3 - kf * _C2
    z = y * y
    s = y + y * z * (_S1 + z * (_S2 + z * _S3))
    c = 1.0 + z * (-0.5 + z * (_K1 + z * (_K2 + z * _K3)))
    qe = kf.astype(jnp.int32) + parity
    r0 = jnp.where((qe & 1) == 0, s, c)
    return jnp.where((qe & 2) == 0, r0, -r0)


def _embed_block(value_ref, chrom_ref, start_ref, end_ref, w_ref, b_ref,
                 denom_ref, table_ref, out_ref):
    v = value_ref[...]
    w = w_ref[...]
    bias = b_ref[...]
    denom = denom_ref[...]
    t = v.shape[0]

    val_emb = v * w + bias

    idx = chrom_ref[...]
    lane = jax.lax.broadcasted_iota(jnp.int32, (t, _V), 1)
    onehot = (idx == lane).astype(jnp.float32)
    chrom_emb = jax.lax.dot_general(
        onehot, table_ref[...], (((1,), (0,)), ((), ())),
        preferred_element_type=jnp.float32)

    parity = jax.lax.broadcasted_iota(jnp.int32, (t, _D), 1) & 1
    pe1 = _pe_interleaved(start_ref[...].astype(jnp.float32), denom, parity)
    pe2 = _pe_interleaved(end_ref[...].astype(jnp.float32), denom, parity)

    out_ref[...] = val_emb + chrom_emb + pe1 + pe2


def kernel(value, chromosome, hg38_start, hg38_end, W, b, chrom_table):
    n = _B * _L
    v2 = value.reshape(n, 1)
    c2 = chromosome.reshape(n, 1).astype(jnp.int32)
    s2 = hg38_start.reshape(n, 1).astype(jnp.int32)
    e2 = hg38_end.reshape(n, 1).astype(jnp.int32)
    wrow = W.reshape(1, _D)
    brow = b.reshape(1, _D)
    _2i = jnp.arange(0, _D, 2, dtype=jnp.float32)
    denom = 10000.0 ** (_2i / _D)
    denom_full = jnp.repeat(denom, 2).reshape(1, _D)

    g = n // _TOK_BLK
    tok = pl.BlockSpec((_TOK_BLK, 1), lambda i: (i, 0))
    row = pl.BlockSpec((1, _D), lambda i: (0, 0))
    out = pl.pallas_call(
        _embed_block,
        grid=(g,),
        in_specs=[tok, tok, tok, tok, row, row, row,
                  pl.BlockSpec((_V, _D), lambda i: (0, 0))],
        out_specs=pl.BlockSpec((_TOK_BLK, _D), lambda i: (i, 0)),
        out_shape=jax.ShapeDtypeStruct((n, _D), jnp.float32),
    )(v2, c2, s2, e2, wrow, brow, denom_full, chrom_table)
    return out.reshape(_B, _L, _D)


# speedup vs baseline: 3.0056x; 1.1636x over previous
import jax
import jax.numpy as jnp
from jax.experimental import pallas as pl
from jax.experimental.pallas import tpu as pltpu

_B, _L, _D, _V = 1024, 200, 128, 25
_TOK_BLK = 2048


_TWO_OVER_PI = 0.6366197723675814
_C1H256 = 402.0
_C1H = 1.5703125
_C1L = 4.8387050628662109375e-4
_C2 = -4.371139000186241e-8
_S1, _S2, _S3 = -1.6666654611e-1, 8.3321608736e-3, -1.9515295891e-4
_K1, _K2, _K3 = 4.166664568298827e-2, -1.388731625493765e-3, 2.443315711809948e-5


def _pe_sum(start_b, end_b, denom, even):
    x = jnp.where(even, start_b, end_b) / denom
    kf = jnp.round(x * _TWO_OVER_PI)
    khf = jnp.floor(kf * (1.0 / 256.0))
    klf = kf - khf * 256.0
    d1 = x - khf * _C1H256
    d2 = d1 - klf * _C1H
    d3 = d2 - kf * _C1L
    y = d3 - kf * _C2
    z = y * y
    s = y + y * z * (_S1 + z * (_S2 + z * _S3))
    c = 1.0 + z * (-0.5 + z * (_K1 + z * (_K2 + z * _K3)))
    ki = kf.astype(jnp.int32)
    qodd = (ki & 1) != 0
    sin_x = jnp.where(qodd, c, s)
    sin_x = jnp.where((ki & 2) == 0, sin_x, -sin_x)
    cos_x = jnp.where(qodd, s, c)
    cos_x = jnp.where(((ki + 1) & 2) == 0, cos_x, -cos_x)
    return jnp.where(even,
                     sin_x + pltpu.roll(sin_x, 127, 1),
                     cos_x + pltpu.roll(cos_x, 1, 1))


def _embed_block(value_ref, chrom_ref, start_ref, end_ref, w_ref, b_ref,
                 denom_ref, table_ref, out_ref):
    v = value_ref[...]
    w = w_ref[...]
    bias = b_ref[...]
    denom = denom_ref[...]
    t = v.shape[0]

    val_emb = v * w + bias

    idx = chrom_ref[...]
    lane = jax.lax.broadcasted_iota(jnp.int32, (t, _V), 1)
    onehot = (idx == lane).astype(jnp.float32)
    chrom_emb = jax.lax.dot_general(
        onehot, table_ref[...], (((1,), (0,)), ((), ())),
        preferred_element_type=jnp.float32)

    even = (jax.lax.broadcasted_iota(jnp.int32, (t, _D), 1) & 1) == 0
    pe = _pe_sum(start_ref[...].astype(jnp.float32),
                 end_ref[...].astype(jnp.float32), denom, even)

    out_ref[...] = val_emb + chrom_emb + pe


def kernel(value, chromosome, hg38_start, hg38_end, W, b, chrom_table):
    n = _B * _L
    v2 = value.reshape(n, 1)
    c2 = chromosome.reshape(n, 1).astype(jnp.int32)
    s2 = hg38_start.reshape(n, 1).astype(jnp.int32)
    e2 = hg38_end.reshape(n, 1).astype(jnp.int32)
    wrow = W.reshape(1, _D)
    brow = b.reshape(1, _D)
    _2i = jnp.arange(0, _D, 2, dtype=jnp.float32)
    denom = 10000.0 ** (_2i / _D)
    denom_full = jnp.repeat(denom, 2).reshape(1, _D)

    g = n // _TOK_BLK
    tok = pl.BlockSpec((_TOK_BLK, 1), lambda i: (i, 0))
    row = pl.BlockSpec((1, _D), lambda i: (0, 0))
    out = pl.pallas_call(
        _embed_block,
        grid=(g,),
        in_specs=[tok, tok, tok, tok, row, row, row,
                  pl.BlockSpec((_V, _D), lambda i: (0, 0))],
        out_specs=pl.BlockSpec((_TOK_BLK, _D), lambda i: (i, 0)),
        out_shape=jax.ShapeDtypeStruct((n, _D), jnp.float32),
    )(v2, c2, s2, e2, wrow, brow, denom_full, chrom_table)
    return out.reshape(_B, _L, _D)


# speedup vs baseline: 3.0902x; 1.0281x over previous
import jax
import jax.numpy as jnp
from jax.experimental import pallas as pl
from jax.experimental.pallas import tpu as pltpu

_B, _L, _D, _V = 1024, 200, 128, 25
_TOK_BLK = 4096


_TWO_OVER_PI = 0.6366197723675814
_C1H256 = 402.0
_C1H = 1.5703125
_C1L = 4.8387050628662109375e-4
_C2 = -4.371139000186241e-8
_S1, _S2, _S3 = -1.6666654611e-1, 8.3321608736e-3, -1.9515295891e-4
_K1, _K2, _K3 = 4.166664568298827e-2, -1.388731625493765e-3, 2.443315711809948e-5


def _pe_sum(start_b, end_b, denom, even):
    x = jnp.where(even, start_b, end_b) / denom
    kf = jnp.round(x * _TWO_OVER_PI)
    khf = jnp.floor(kf * (1.0 / 256.0))
    klf = kf - khf * 256.0
    d1 = x - khf * _C1H256
    d2 = d1 - klf * _C1H
    d3 = d2 - kf * _C1L
    y = d3 - kf * _C2
    z = y * y
    s = y + y * z * (_S1 + z * (_S2 + z * _S3))
    c = 1.0 + z * (-0.5 + z * (_K1 + z * (_K2 + z * _K3)))
    ki = kf.astype(jnp.int32)
    qodd = (ki & 1) != 0
    sin_x = jnp.where(qodd, c, s)
    sin_x = jnp.where((ki & 2) == 0, sin_x, -sin_x)
    cos_x = jnp.where(qodd, s, c)
    cos_x = jnp.where(((ki + 1) & 2) == 0, cos_x, -cos_x)
    return jnp.where(even,
                     sin_x + pltpu.roll(sin_x, 127, 1),
                     cos_x + pltpu.roll(cos_x, 1, 1))


def _embed_block(value_ref, chrom_ref, start_ref, end_ref, w_ref, b_ref,
                 denom_ref, table_ref, out_ref):
    v = value_ref[...]
    w = w_ref[...]
    bias = b_ref[...]
    denom = denom_ref[...]
    t = v.shape[0]

    val_emb = v * w + bias

    idx = chrom_ref[...]
    lane = jax.lax.broadcasted_iota(jnp.int32, (t, _V), 1)
    onehot = (idx == lane).astype(jnp.float32)
    chrom_emb = jax.lax.dot_general(
        onehot, table_ref[...], (((1,), (0,)), ((), ())),
        preferred_element_type=jnp.float32)

    out_ref[...] = val_emb + chrom_emb

    even = (jax.lax.broadcasted_iota(jnp.int32, (t, _D), 1) & 1) == 0
    pe = _pe_sum(start_ref[...].astype(jnp.float32),
                 end_ref[...].astype(jnp.float32), denom, even)

    out_ref[...] += pe


def kernel(value, chromosome, hg38_start, hg38_end, W, b, chrom_table):
    n = _B * _L
    v2 = value.reshape(n, 1)
    c2 = chromosome.reshape(n, 1).astype(jnp.int32)
    s2 = hg38_start.reshape(n, 1).astype(jnp.int32)
    e2 = hg38_end.reshape(n, 1).astype(jnp.int32)
    wrow = W.reshape(1, _D)
    brow = b.reshape(1, _D)
    _2i = jnp.arange(0, _D, 2, dtype=jnp.float32)
    denom = 10000.0 ** (_2i / _D)
    denom_full = jnp.repeat(denom, 2).reshape(1, _D)

    g = n // _TOK_BLK
    tok = pl.BlockSpec((_TOK_BLK, 1), lambda i: (i, 0))
    row = pl.BlockSpec((1, _D), lambda i: (0, 0))
    out = pl.pallas_call(
        _embed_block,
        grid=(g,),
        in_specs=[tok, tok, tok, tok, row, row, row,
                  pl.BlockSpec((_V, _D), lambda i: (0, 0))],
        out_specs=pl.BlockSpec((_TOK_BLK, _D), lambda i: (i, 0)),
        out_shape=jax.ShapeDtypeStruct((n, _D), jnp.float32),
    )(v2, c2, s2, e2, wrow, brow, denom_full, chrom_table)
    return out.reshape(_B, _L, _D)
